# baseline (speedup 1.0000x reference)
import functools
import math

import jax
import jax.numpy as jnp
from jax import lax
from jax.experimental import pallas as pl
from jax.experimental.pallas import tpu as pltpu
from jax.experimental.pallas import tpu_sc as plsc

N_ATOMS = 100000
N_MOL = 4096
SQ3 = 1.0 / math.sqrt(3.0)

BLK = 3584
N_PAD = 100352
GRID = N_PAD // BLK

NC = 2
NS = 16
NW = NC * NS
CHUNK = N_PAD // NW
ROWS_PER_TILE = N_MOL // NS


def _atom_body(xs_ref, x0_ref, xa_ref, xb_ref, sw1_ref, sb1_ref, sw2_ref,
               sb2_ref, pw0_ref, pb0_ref, w2a_ref, w2b_ref, s_ref, st_ref,
               qw0_ref, qb0_ref, q2_ref, out_ref):
    h = xs_ref[...] @ sw1_ref[...] + sb1_ref[...]
    h = h * jax.nn.sigmoid(h)
    so = h @ sw2_ref[...] + sb2_ref[...]
    h0 = x0_ref[...] @ pw0_ref[...] + pb0_ref[...]
    h0 = h0 * jax.nn.sigmoid(jnp.abs(h0))
    lane = lax.broadcasted_iota(jnp.int32, (BLK, 128), 1)
    xb = jnp.where(lane < 96, xb_ref[...], 0.0)
    h2 = xa_ref[...] @ w2a_ref[...] + xb @ w2b_ref[...]
    n2 = jnp.sqrt((h2 * h2) @ s_ref[...] + 1e-12)
    h2 = h2 * (jax.nn.sigmoid(n2) @ st_ref[...])
    o0 = h0 @ qw0_ref[...] * 0.125 + qb0_ref[...]
    o2 = h2 @ q2_ref[...]
    a6 = jnp.concatenate([o0 * so[:, 0:1], o2 * so[:, 1:2]], axis=1)
    row = pl.program_id(0) * BLK + lax.broadcasted_iota(jnp.int32, (BLK, 1), 0)
    a6 = jnp.where(row < N_ATOMS, a6, 0.0)
    out_ref[:, 0:6] = a6
    out_ref[:, 6:8] = jnp.zeros((BLK, 2), jnp.float32)


_ATOM_IN_SPECS = [
    pl.BlockSpec((BLK, 128), lambda i: (i, 0)),
    pl.BlockSpec((BLK, 128), lambda i: (i, 0)),
    pl.BlockSpec((BLK, 128), lambda i: (i, 2)),
    pl.BlockSpec((BLK, 128), lambda i: (i, 3)),
    pl.BlockSpec((128, 64), lambda i: (0, 0)),
    pl.BlockSpec((1, 64), lambda i: (0, 0)),
    pl.BlockSpec((64, 2), lambda i: (0, 0)),
    pl.BlockSpec((1, 2), lambda i: (0, 0)),
    pl.BlockSpec((128, 64), lambda i: (0, 0)),
    pl.BlockSpec((1, 64), lambda i: (0, 0)),
    pl.BlockSpec((128, 80), lambda i: (0, 0)),
    pl.BlockSpec((128, 80), lambda i: (0, 0)),
    pl.BlockSpec((80, 16), lambda i: (0, 0)),
    pl.BlockSpec((16, 80), lambda i: (0, 0)),
    pl.BlockSpec((64, 1), lambda i: (0, 0)),
    pl.BlockSpec((1, 1), lambda i: (0, 0)),
    pl.BlockSpec((80, 5), lambda i: (0, 0)),
]
_ATOM_OUT_SPEC = pl.BlockSpec((BLK, 8), lambda i: (i, 0))


def _post_body(parts_ref, sel_ref, mmat_ref, out_ref):
    m = parts_ref[0:N_MOL, :] + parts_ref[N_MOL:2 * N_MOL, :]
    dsq = (m * m) @ sel_ref[...]
    dn = jnp.sqrt(dsq + 1e-12)
    lane = lax.broadcasted_iota(jnp.int32, (N_MOL, 8), 1)
    m2 = jnp.where(lane == 6, jnp.broadcast_to(dn, (N_MOL, 8)), m)
    out_ref[...] = m2 @ mmat_ref[...]


def _sc_scatter_body(atom_hbm, bidx_hbm, zero_hbm, out_hbm, idx_v, rows_v,
                     acc):
    c = lax.axis_index("c")
    s = lax.axis_index("s")
    base = (s * NC + c) * CHUNK

    pltpu.sync_copy(zero_hbm.at[pl.ds(s * ROWS_PER_TILE, ROWS_PER_TILE)],
                    acc.at[pl.ds(s * ROWS_PER_TILE, ROWS_PER_TILE)])
    plsc.subcore_barrier()
    pltpu.sync_copy(bidx_hbm.at[pl.ds(base, CHUNK)], idx_v)
    pltpu.sync_copy(atom_hbm.at[pl.ds(base, CHUNK)], rows_v)
    pltpu.sync_copy(rows_v, acc.at[idx_v], add=True)
    plsc.subcore_barrier()
    pltpu.sync_copy(acc.at[pl.ds(s * ROWS_PER_TILE, ROWS_PER_TILE)],
                    out_hbm.at[pl.ds(c * N_MOL + s * ROWS_PER_TILE,
                                     ROWS_PER_TILE)])


@functools.cache
def _sc_scatter():
    return functools.partial(
        pl.kernel,
        out_type=jax.ShapeDtypeStruct((NC * N_MOL, 8), jnp.float32),
        mesh=plsc.VectorSubcoreMesh(core_axis_name="c",
                                    subcore_axis_name="s"),
        scratch_types=[
            pltpu.VMEM((CHUNK,), jnp.int32),
            pltpu.VMEM((CHUNK, 8), jnp.float32),
            pltpu.VMEM_SHARED((N_MOL, 8), jnp.float32),
        ],
        compiler_params=pltpu.CompilerParams(use_tc_tiling_on_sc=False),
    )(_sc_scatter_body)


def kernel(x_scalar, x_spherical, coord, batch_index, sw1, sb1, sw2, sb2,
           pw0, pb0, pw2, qw0, qb0, qw2):
    f32 = jnp.float32
    eye5 = jnp.eye(5, dtype=f32)
    w2 = jnp.kron(pw2, eye5) * (1.0 / math.sqrt(32.0))
    w2a = jnp.zeros((128, 80), f32).at[64:128].set(w2[0:64])
    w2b = jnp.zeros((128, 80), f32).at[0:96].set(w2[64:160])
    q2 = jnp.kron(qw2, eye5) * (1.0 / math.sqrt(16.0))
    s_sum = jnp.kron(jnp.eye(16, dtype=f32), jnp.ones((5, 1), f32))
    pw0s = pw0 * (1.0 / math.sqrt(128.0))

    atom = pl.pallas_call(
        _atom_body,
        grid=(GRID,),
        in_specs=_ATOM_IN_SPECS,
        out_specs=_ATOM_OUT_SPEC,
        out_shape=jax.ShapeDtypeStruct((N_PAD, 8), f32),
        compiler_params=pltpu.CompilerParams(
            dimension_semantics=("arbitrary",)),
    )(x_scalar, x_spherical, x_spherical, x_spherical, sw1,
      sb1.reshape(1, 64), sw2, sb2.reshape(1, 2), pw0s, pb0.reshape(1, 64),
      w2a, w2b, s_sum, s_sum.T, qw0, qb0.reshape(1, 1), q2)

    bidx = jnp.concatenate(
        [batch_index.astype(jnp.int32),
         jnp.zeros((N_PAD - N_ATOMS,), jnp.int32)])
    parts = _sc_scatter()(atom, bidx, jnp.zeros((N_MOL, 8), f32))

    sel = jnp.zeros((8, 1), f32).at[1:6, 0].set(1.0)
    mmat = jnp.zeros((8, 9), f32)
    mmat = mmat.at[0, [0, 4, 8]].set(1.0)
    mmat = mmat.at[1, [1, 3]].set(1.0)
    mmat = mmat.at[2, [5, 7]].set(1.0)
    mmat = mmat.at[3, [0, 4]].set(-SQ3).at[3, 8].set(2.0 * SQ3)
    mmat = mmat.at[4, [2, 6]].set(1.0)
    mmat = mmat.at[5, 0].set(1.0).at[5, 4].set(-1.0)
    mmat = mmat.at[6, [0, 4, 8]].set(SQ3)
    out9 = pl.pallas_call(
        _post_body,
        out_shape=jax.ShapeDtypeStruct((N_MOL, 9), f32),
    )(parts, sel, mmat)
    return out9.reshape(N_MOL, 3, 3)


# speedup vs baseline: 2.9730x; 2.9730x over previous
import functools
import math

import jax
import jax.numpy as jnp
from jax import lax
from jax.experimental import pallas as pl
from jax.experimental.pallas import tpu as pltpu
from jax.experimental.pallas import tpu_sc as plsc

N_ATOMS = 100000
N_MOL = 4096
SQ3 = 1.0 / math.sqrt(3.0)

BLK = 4096
N_PAD = 102400
GRID = N_PAD // BLK
SLABS = N_PAD // 128
SLABS_PER_BLK = BLK // 128

NC = 2
NS = 16
NW = NC * NS
APT = N_PAD // NW
SPT = SLABS // NW
TAIL = N_ATOMS - (NW - 1) * APT
ROWS_PER_TILE = N_MOL // NS


def _atom_body(xs_ref, x0t_ref, x2t_ref, sw1_ref, sb1_ref, sw2_ref,
               sb2_ref, pw0t_ref, pb0t_ref, w2t_ref, st_ref, s_ref,
               qw0_ref, qb0_ref, q2_ref, out_ref):
    h = xs_ref[...] @ sw1_ref[...] + sb1_ref[...]
    h = h * jax.nn.sigmoid(h)
    dn01 = (((0,), (1,)), ((), ()))
    sot = lax.dot_general(sw2_ref[...], h, dn01) + sb2_ref[...]
    h0t = pw0t_ref[...] @ x0t_ref[...] + pb0t_ref[...]
    h0t = h0t * jax.nn.sigmoid(jnp.abs(h0t))
    h2t = w2t_ref[...] @ x2t_ref[...]
    n2t = jnp.sqrt(st_ref[...] @ (h2t * h2t) + 1e-12)
    h2t = h2t * (s_ref[...] @ jax.nn.sigmoid(n2t))
    dn0 = (((0,), (0,)), ((), ()))
    o0t = lax.dot_general(qw0_ref[...], h0t, dn0) * 0.125 + qb0_ref[...]
    o2t = lax.dot_general(q2_ref[...], h2t, dn0)
    a8t = jnp.concatenate(
        [o0t * sot[0:1, :], o2t * sot[1:2, :],
         jnp.zeros((2, BLK), jnp.float32)], axis=0)
    col = pl.program_id(0) * BLK + lax.broadcasted_iota(jnp.int32, (8, BLK), 1)
    a8t = jnp.where(col < N_ATOMS, a8t, 0.0)
    for c in range(SLABS_PER_BLK):
        out_ref[c] = a8t[:, c * 128:(c + 1) * 128]


_ATOM_IN_SPECS = [
    pl.BlockSpec((BLK, 128), lambda i: (i, 0)),
    pl.BlockSpec((128, BLK), lambda i: (0, i)),
    pl.BlockSpec((160, BLK), lambda i: (2, i)),
    pl.BlockSpec((128, 64), lambda i: (0, 0)),
    pl.BlockSpec((1, 64), lambda i: (0, 0)),
    pl.BlockSpec((64, 2), lambda i: (0, 0)),
    pl.BlockSpec((2, 1), lambda i: (0, 0)),
    pl.BlockSpec((64, 128), lambda i: (0, 0)),
    pl.BlockSpec((64, 1), lambda i: (0, 0)),
    pl.BlockSpec((80, 160), lambda i: (0, 0)),
    pl.BlockSpec((16, 80), lambda i: (0, 0)),
    pl.BlockSpec((80, 16), lambda i: (0, 0)),
    pl.BlockSpec((64, 1), lambda i: (0, 0)),
    pl.BlockSpec((1, 1), lambda i: (0, 0)),
    pl.BlockSpec((80, 5), lambda i: (0, 0)),
]
_ATOM_OUT_SPEC = pl.BlockSpec((SLABS_PER_BLK, 8, 128), lambda i: (i, 0, 0))


def _post_body(parts_ref, sel_ref, mmat_ref, out_ref):
    m = parts_ref[0:N_MOL, :] + parts_ref[N_MOL:2 * N_MOL, :]
    dsq = (m * m) @ sel_ref[...]
    dn = jnp.sqrt(dsq + 1e-12)
    lane = lax.broadcasted_iota(jnp.int32, (N_MOL, 8), 1)
    m2 = jnp.where(lane == 6, jnp.broadcast_to(dn, (N_MOL, 8)), m)
    out_ref[...] = m2 @ mmat_ref[...]


def _sc_scatter_body(atom_hbm, bidx_hbm, zero_hbm, out_hbm, idx_v, slab_v,
                     rows_v, acc):
    c = lax.axis_index("c")
    s = lax.axis_index("s")
    wid = s * NC + c
    abase = wid * APT

    pltpu.sync_copy(zero_hbm.at[pl.ds(s * ROWS_PER_TILE, ROWS_PER_TILE)],
                    acc.at[pl.ds(s * ROWS_PER_TILE, ROWS_PER_TILE)])
    pltpu.sync_copy(atom_hbm.at[pl.ds(wid * SPT, SPT)], slab_v)

    @pl.when(wid < NW - 1)
    def _():
        pltpu.sync_copy(bidx_hbm.at[pl.ds(abase, APT)], idx_v)

    @pl.when(wid == NW - 1)
    def _():
        pltpu.sync_copy(bidx_hbm.at[pl.ds((NW - 1) * APT, TAIL)],
                        idx_v.at[pl.ds(0, TAIL)])
        zeros16 = jnp.zeros((16,), jnp.int32)

        def _zr(j, carry):
            idx_v[pl.ds(TAIL + j * 16, 16)] = zeros16
            return carry

        lax.fori_loop(0, (APT - TAIL) // 16, _zr, 0)

    iota16 = lax.iota(jnp.int32, 16)

    def _rp(g, carry):
        slab = g // 8
        l16 = (g % 8) * 16
        rowi = g * 16 + iota16
        for r in range(8):
            vals = slab_v[slab, r, pl.ds(l16, 16)]
            plsc.store_scatter(rows_v, [rowi, jnp.full((16,), r, jnp.int32)],
                               vals)
        return carry

    lax.fori_loop(0, APT // 16, _rp, 0)

    plsc.subcore_barrier()
    pltpu.sync_copy(rows_v, acc.at[idx_v], add=True)
    plsc.subcore_barrier()
    pltpu.sync_copy(acc.at[pl.ds(s * ROWS_PER_TILE, ROWS_PER_TILE)],
                    out_hbm.at[pl.ds(c * N_MOL + s * ROWS_PER_TILE,
                                     ROWS_PER_TILE)])


@functools.cache
def _sc_scatter():
    return functools.partial(
        pl.kernel,
        out_type=jax.ShapeDtypeStruct((NC * N_MOL, 8), jnp.float32),
        mesh=plsc.VectorSubcoreMesh(core_axis_name="c",
                                    subcore_axis_name="s"),
        scratch_types=[
            pltpu.VMEM((APT,), jnp.int32),
            pltpu.VMEM((SPT, 8, 128), jnp.float32),
            pltpu.VMEM((APT, 8), jnp.float32),
            pltpu.VMEM_SHARED((N_MOL, 8), jnp.float32),
        ],
        compiler_params=pltpu.CompilerParams(use_tc_tiling_on_sc=False,
                                             needs_layout_passes=False),
    )(_sc_scatter_body)


def kernel(x_scalar, x_spherical, coord, batch_index, sw1, sb1, sw2, sb2,
           pw0, pb0, pw2, qw0, qb0, qw2):
    f32 = jnp.float32
    eye5 = jnp.eye(5, dtype=f32)
    w2 = jnp.kron(pw2, eye5) * (1.0 / math.sqrt(32.0))
    q2 = jnp.kron(qw2, eye5) * (1.0 / math.sqrt(16.0))
    s_sum = jnp.kron(jnp.eye(16, dtype=f32), jnp.ones((5, 1), f32))
    pw0s_t = pw0.T * (1.0 / math.sqrt(128.0))
    xsph_t = x_spherical.T

    atom = pl.pallas_call(
        _atom_body,
        grid=(GRID,),
        in_specs=_ATOM_IN_SPECS,
        out_specs=_ATOM_OUT_SPEC,
        out_shape=jax.ShapeDtypeStruct((SLABS, 8, 128), f32),
        compiler_params=pltpu.CompilerParams(
            dimension_semantics=("arbitrary",)),
    )(x_scalar, xsph_t, xsph_t, sw1, sb1.reshape(1, 64), sw2,
      sb2.reshape(2, 1), pw0s_t, pb0.reshape(64, 1), w2.T, s_sum.T, s_sum,
      qw0, qb0.reshape(1, 1), q2)

    bidx = batch_index.astype(jnp.int32)
    parts = _sc_scatter()(atom, bidx, jnp.zeros((N_MOL, 8), f32))

    sel = jnp.zeros((8, 1), f32).at[1:6, 0].set(1.0)
    mmat = jnp.zeros((8, 9), f32)
    mmat = mmat.at[0, [0, 4, 8]].set(1.0)
    mmat = mmat.at[1, [1, 3]].set(1.0)
    mmat = mmat.at[2, [5, 7]].set(1.0)
    mmat = mmat.at[3, [0, 4]].set(-SQ3).at[3, 8].set(2.0 * SQ3)
    mmat = mmat.at[4, [2, 6]].set(1.0)
    mmat = mmat.at[5, 0].set(1.0).at[5, 4].set(-1.0)
    mmat = mmat.at[6, [0, 4, 8]].set(SQ3)
    out9 = pl.pallas_call(
        _post_body,
        out_shape=jax.ShapeDtypeStruct((N_MOL, 9), f32),
    )(parts, sel, mmat)
    return out9.reshape(N_MOL, 3, 3)
